# baseline (speedup 1.0000x reference)
import jax
import jax.numpy as jnp
from jax import lax
from jax.experimental import pallas as pl
from jax.experimental.pallas import tpu as pltpu
from jax.experimental.pallas import tpu_sc as plsc

N_NODE = 10000
E = 160000
D = 128
NC = 2
NS = 16
NW = NC * NS
EP = 165888
CK = 96
CHUNKS = EP // (NW * CK)
EPT = EP // NW
NPAD = 10240
RPT = NPAD // NS
CROWS = NPAD // 128


def _mm_body(fu, fi, wf, bf, wc, bc, wp, bp, of, oc, op):
    of[...] = jnp.dot(fu[...], wf[...], preferred_element_type=jnp.float32) + bf[...]
    oc[...] = jnp.dot(fu[...], wc[...], preferred_element_type=jnp.float32) + bc[...]
    op[...] = jnp.dot(fi[...], wp[...], preferred_element_type=jnp.float32) + bp[...]


def _matmuls(feat_user, feat_item, Wf, bf, Wc, bc, Wp, bp):
    BR = 1000
    grid = (N_NODE // BR,)
    row_spec = pl.BlockSpec((BR, D), lambda i: (i, 0))
    w_spec = pl.BlockSpec((D, D), lambda i: (0, 0))
    b_spec = pl.BlockSpec((1, D), lambda i: (0, 0))
    out = jax.ShapeDtypeStruct((N_NODE, D), jnp.float32)
    return pl.pallas_call(
        _mm_body,
        grid=grid,
        in_specs=[row_spec, row_spec, w_spec, b_spec, w_spec, b_spec,
                  w_spec, b_spec],
        out_specs=[row_spec, row_spec, row_spec],
        out_shape=[out, out, out],
    )(feat_user, feat_item, Wf, bf.reshape(1, D), Wc, bc.reshape(1, D),
      Wp, bp.reshape(1, D))


NB = 2
GROUPS = CHUNKS // NB


def _sc_body(whf, whc, whp, s0, d0, s1, d1, s2, d2, zrows, zcnt,
             out_sum, oc0, oc1, oc2, ssum, sidx, didx, r0, r1,
             cnt, gsem, ssem):
    cid = lax.axis_index("c")
    sid = lax.axis_index("s")
    wid = cid * NS + sid
    ones = jnp.full((16,), 1.0, jnp.float32)
    rows = [r0, r1]

    for t, (wh, se, de) in enumerate(
            [(whf, s0, d0), (whc, s1, d1), (whp, s2, d2)]):
        pltpu.sync_copy(zcnt, cnt)
        pltpu.sync_copy(zrows, ssum.at[pl.ds(sid * RPT, RPT)])
        base = pl.multiple_of(wid * EPT, 8)
        pltpu.sync_copy(se.at[pl.ds(base, EPT)], sidx)
        pltpu.sync_copy(de.at[wid], didx)
        plsc.subcore_barrier()

        def gather(c, b):
            return pltpu.make_async_copy(
                wh.at[sidx.at[pl.ds(c * CK, CK)]], rows[b], gsem)

        def scat(c, b):
            return pltpu.make_async_copy(rows[b], ssum.at[didx.at[c]], ssem)

        gather(0, 0).start()

        def group(g, carry):
            for b in range(NB):
                c = g * NB + b
                nb = (b + 1) % NB
                if b == NB - 1:
                    @pl.when(g < GROUPS - 1)
                    def _():
                        gather(c + 1, nb).start()
                else:
                    gather(c + 1, nb).start()
                gather(c, b).wait()
                s = scat(c, b)
                s.start(add=True)
                for j in range(CK // 16):
                    d16 = didx[c, pl.ds(j * 16, 16)]
                    plsc.addupdate_scatter(cnt, [d16], ones)
                s.wait()
            return carry

        lax.fori_loop(0, GROUPS, group, 0)
        plsc.subcore_barrier()
        pltpu.sync_copy(ssum.at[pl.ds(sid * RPT, RPT)],
                        out_sum.at[t, cid, pl.ds(sid * RPT, RPT)])
        oc = [oc0, oc1, oc2][t]
        pltpu.sync_copy(cnt, oc.at[pl.ds(wid * NPAD, NPAD)])


def _sc_scatter(whf, whc, whp, edges):
    (s0, d0), (s1, d1), (s2, d2) = edges
    zrows = jnp.zeros((RPT, D), jnp.float32)
    zcnt = jnp.zeros((NPAD,), jnp.float32)
    mesh = plsc.VectorSubcoreMesh(core_axis_name="c", subcore_axis_name="s")
    call = pl.kernel(
        _sc_body,
        out_type=(jax.ShapeDtypeStruct((3, NC, NPAD, D), jnp.float32),
                  jax.ShapeDtypeStruct((NW * NPAD,), jnp.float32),
                  jax.ShapeDtypeStruct((NW * NPAD,), jnp.float32),
                  jax.ShapeDtypeStruct((NW * NPAD,), jnp.float32)),
        mesh=mesh,
        compiler_params=pltpu.CompilerParams(needs_layout_passes=False),
        scratch_types=[
            pltpu.VMEM_SHARED((NPAD, D), jnp.float32),
            pltpu.VMEM((EPT,), jnp.int32),
            pltpu.VMEM((CHUNKS, CK), jnp.int32),
            pltpu.VMEM((CK, D), jnp.float32),
            pltpu.VMEM((CK, D), jnp.float32),
            pltpu.VMEM((NPAD,), jnp.float32),
            pltpu.SemaphoreType.DMA,
            pltpu.SemaphoreType.DMA,
        ],
    )
    return call(whf, whc, whp, s0, d0, s1, d1, s2, d2, zrows, zcnt)


def _norm_body(s_ref, c_ref, hu_ref, hi_ref):
    s = s_ref[...]
    c = jnp.sum(c_ref[...], axis=-1)
    c = c.reshape(3, -1)
    cm = jnp.maximum(c, 1.0)
    hu_ref[...] = ((s[0, 0] + s[0, 1]) / cm[0, :, None]
                   + (s[2, 0] + s[2, 1]) / cm[2, :, None])
    hi_ref[...] = (s[1, 0] + s[1, 1]) / cm[1, :, None]


def _normalize(sums, cnts):
    BR = RPT
    grid = (NPAD // BR,)
    out = jax.ShapeDtypeStruct((NPAD, D), jnp.float32)
    return pl.pallas_call(
        _norm_body,
        grid=grid,
        in_specs=[pl.BlockSpec((3, NC, BR, D), lambda i: (0, 0, i, 0)),
                  pl.BlockSpec((3, BR // 128, 128, NW), lambda i: (0, i, 0, 0))],
        out_specs=[pl.BlockSpec((BR, D), lambda i: (i, 0)),
                   pl.BlockSpec((BR, D), lambda i: (i, 0))],
        out_shape=[out, out],
    )(sums, cnts)


def _prep_edges(e):
    pad = EP - E
    src = jnp.concatenate([e[0].astype(jnp.int32),
                           jnp.zeros((pad,), jnp.int32)])
    junk = N_NODE + jnp.arange(pad, dtype=jnp.int32) % (NPAD - N_NODE)
    dst = jnp.concatenate([e[1].astype(jnp.int32), junk])
    return src, dst.reshape(NW, CHUNKS, CK)


def kernel(feat_user, feat_item, edge_follows, edge_clicks, edge_purchased,
           W_follows, b_follows, W_clicks, b_clicks, W_purchased, b_purchased):
    whf, whc, whp = _matmuls(feat_user, feat_item, W_follows, b_follows,
                             W_clicks, b_clicks, W_purchased, b_purchased)
    edges = [_prep_edges(edge_follows), _prep_edges(edge_clicks),
             _prep_edges(edge_purchased)]
    sums, oc0, oc1, oc2 = _sc_scatter(whf, whc, whp, edges)
    cnts = jnp.stack([oc0, oc1, oc2]).reshape(3, NW, CROWS, 128)
    hu, hi = _normalize(sums, cnts.transpose(0, 2, 3, 1))
    return hu[:N_NODE], hi[:N_NODE]


# speedup vs baseline: 1.0739x; 1.0739x over previous
import jax
import jax.numpy as jnp
from jax import lax
from jax.experimental import pallas as pl
from jax.experimental.pallas import tpu as pltpu
from jax.experimental.pallas import tpu_sc as plsc

N_NODE = 10000
E = 160000
D = 128
NC = 2
NS = 16
NW = NC * NS
EP = 165888
CK = 96
CHUNKS = EP // (NW * CK)
EPT = EP // NW
NPAD = 10240
RPT = NPAD // NS
CROWS = NPAD // 128


NB = 2
GROUPS = CHUNKS // NB


def _sc_body(fu, fi, s0, d0, s1, d1, s2, d2, zrows, zcnt,
             out_sum, oc0, oc1, oc2, ssum, sidx, didx, r0, r1,
             cnt, gsem, ssem):
    cid = lax.axis_index("c")
    sid = lax.axis_index("s")
    wid = cid * NS + sid
    ones = jnp.full((16,), 1.0, jnp.float32)
    rows = [r0, r1]

    for t, (tab, se, de) in enumerate(
            [(fu, s0, d0), (fu, s1, d1), (fi, s2, d2)]):
        pltpu.sync_copy(zcnt, cnt)
        pltpu.sync_copy(zrows, ssum.at[pl.ds(sid * RPT, RPT)])
        base = pl.multiple_of(wid * EPT, 8)
        pltpu.sync_copy(se.at[pl.ds(base, EPT)], sidx)
        pltpu.sync_copy(de.at[wid], didx)
        plsc.subcore_barrier()

        def gather(c, b):
            return pltpu.make_async_copy(
                tab.at[sidx.at[pl.ds(c * CK, CK)]], rows[b], gsem)

        def scat(c, b):
            return pltpu.make_async_copy(rows[b], ssum.at[didx.at[c]], ssem)

        gather(0, 0).start()

        def group(g, carry):
            for b in range(NB):
                c = g * NB + b
                nb = (b + 1) % NB
                if b == NB - 1:
                    @pl.when(g < GROUPS - 1)
                    def _():
                        gather(c + 1, nb).start()
                else:
                    gather(c + 1, nb).start()
                gather(c, b).wait()
                s = scat(c, b)
                s.start(add=True)
                for j in range(CK // 16):
                    d16 = didx[c, pl.ds(j * 16, 16)]
                    plsc.addupdate_scatter(cnt, [d16], ones)
                s.wait()
            return carry

        lax.fori_loop(0, GROUPS, group, 0)
        plsc.subcore_barrier()
        pltpu.sync_copy(ssum.at[pl.ds(sid * RPT, RPT)],
                        out_sum.at[t, cid, pl.ds(sid * RPT, RPT)])
        oc = [oc0, oc1, oc2][t]
        pltpu.sync_copy(cnt, oc.at[pl.ds(wid * NPAD, NPAD)])


def _sc_scatter(feat_user, feat_item, edges):
    (s0, d0), (s1, d1), (s2, d2) = edges
    zrows = jnp.zeros((RPT, D), jnp.float32)
    zcnt = jnp.zeros((NPAD,), jnp.float32)
    mesh = plsc.VectorSubcoreMesh(core_axis_name="c", subcore_axis_name="s")
    call = pl.kernel(
        _sc_body,
        out_type=(jax.ShapeDtypeStruct((3, NC, NPAD, D), jnp.float32),
                  jax.ShapeDtypeStruct((NW * NPAD,), jnp.float32),
                  jax.ShapeDtypeStruct((NW * NPAD,), jnp.float32),
                  jax.ShapeDtypeStruct((NW * NPAD,), jnp.float32)),
        mesh=mesh,
        compiler_params=pltpu.CompilerParams(needs_layout_passes=False),
        scratch_types=[
            pltpu.VMEM_SHARED((NPAD, D), jnp.float32),
            pltpu.VMEM((EPT,), jnp.int32),
            pltpu.VMEM((CHUNKS, CK), jnp.int32),
            pltpu.VMEM((CK, D), jnp.float32),
            pltpu.VMEM((CK, D), jnp.float32),
            pltpu.VMEM((NPAD,), jnp.float32),
            pltpu.SemaphoreType.DMA,
            pltpu.SemaphoreType.DMA,
        ],
    )
    return call(feat_user, feat_item, s0, d0, s1, d1, s2, d2, zrows, zcnt)


def _comb_body(s_ref, c_ref, wf, bf, wc, bc, wp, bp, hu_ref, hi_ref):
    s = s_ref[...]
    c = jnp.sum(c_ref[...], axis=-1)
    c = c.reshape(3, -1)
    cm = jnp.maximum(c, 1.0)
    mf = (s[0, 0] + s[0, 1]) / cm[0, :, None]
    mc = (s[1, 0] + s[1, 1]) / cm[1, :, None]
    mp = (s[2, 0] + s[2, 1]) / cm[2, :, None]
    hu_ref[...] = (jnp.dot(mf, wf[...], preferred_element_type=jnp.float32)
                   + jnp.dot(mp, wp[...], preferred_element_type=jnp.float32)
                   + jnp.where(c[0, :, None] > 0, bf[...], 0.0)
                   + jnp.where(c[2, :, None] > 0, bp[...], 0.0))
    hi_ref[...] = (jnp.dot(mc, wc[...], preferred_element_type=jnp.float32)
                   + jnp.where(c[1, :, None] > 0, bc[...], 0.0))


def _combine(sums, cnts, Wf, bf, Wc, bc, Wp, bp):
    BR = RPT
    grid = (NPAD // BR,)
    w_spec = pl.BlockSpec((D, D), lambda i: (0, 0))
    b_spec = pl.BlockSpec((1, D), lambda i: (0, 0))
    out = jax.ShapeDtypeStruct((NPAD, D), jnp.float32)
    return pl.pallas_call(
        _comb_body,
        grid=grid,
        in_specs=[pl.BlockSpec((3, NC, BR, D), lambda i: (0, 0, i, 0)),
                  pl.BlockSpec((3, BR // 128, 128, NW), lambda i: (0, i, 0, 0)),
                  w_spec, b_spec, w_spec, b_spec, w_spec, b_spec],
        out_specs=[pl.BlockSpec((BR, D), lambda i: (i, 0)),
                   pl.BlockSpec((BR, D), lambda i: (i, 0))],
        out_shape=[out, out],
    )(sums, cnts, Wf, bf.reshape(1, D), Wc, bc.reshape(1, D),
      Wp, bp.reshape(1, D))


def _prep_edges(e):
    pad = EP - E
    src = jnp.concatenate([e[0].astype(jnp.int32),
                           jnp.zeros((pad,), jnp.int32)])
    junk = N_NODE + jnp.arange(pad, dtype=jnp.int32) % (NPAD - N_NODE)
    dst = jnp.concatenate([e[1].astype(jnp.int32), junk])
    return src, dst.reshape(NW, CHUNKS, CK)


def kernel(feat_user, feat_item, edge_follows, edge_clicks, edge_purchased,
           W_follows, b_follows, W_clicks, b_clicks, W_purchased, b_purchased):
    edges = [_prep_edges(edge_follows), _prep_edges(edge_clicks),
             _prep_edges(edge_purchased)]
    sums, oc0, oc1, oc2 = _sc_scatter(feat_user, feat_item, edges)
    cnts = jnp.stack([oc0, oc1, oc2]).reshape(3, NW, CROWS, 128)
    hu, hi = _combine(sums, cnts.transpose(0, 2, 3, 1), W_follows, b_follows,
                      W_clicks, b_clicks, W_purchased, b_purchased)
    return hu[:N_NODE], hi[:N_NODE]
